# baseline (speedup 1.0000x reference)
import functools

import jax
import jax.numpy as jnp
from jax import lax
from jax.experimental import pallas as pl
from jax.experimental.pallas import tpu as pltpu
from jax.experimental.pallas import tpu_sc as plsc

B, L, NVARS = 4096, 200, 2
VOCAB, DIM = 1000000, 32
N = B * L

_info = plsc.get_sparse_core_info()
NC, NS = _info.num_cores, _info.num_subcores
NW = NC * NS
BT = B // 128
CL = 4
GB = L // CL
G2 = GB // 2

_mesh = plsc.VectorSubcoreMesh(core_axis_name="c", subcore_axis_name="s")


@functools.partial(
    pl.kernel,
    out_type=jax.ShapeDtypeStruct((L, 8, BT, 1024), jnp.float32),
    mesh=_mesh,
    scratch_types=[
        pltpu.VMEM((2, CL, 128), jnp.int32),
        pltpu.VMEM((2, CL, 128), jnp.int32),
        pltpu.VMEM((2, CL, 128, DIM), jnp.float32),
        pltpu.VMEM((2, CL, 128, DIM), jnp.float32),
        pltpu.VMEM((2, 8192), jnp.float32),
        pltpu.SemaphoreType.DMA,
        pltpu.SemaphoreType.DMA,
        pltpu.SemaphoreType.DMA,
        pltpu.SemaphoreType.DMA,
    ],
    compiler_params=pltpu.CompilerParams(use_tc_tiling_on_sc=False,
                                         needs_layout_passes=False),
)
def _embed_sc(idx_hbm, w0_hbm, w1_hbm, out_hbm,
              idx0_v, idx1_v, rows0_v, rows1_v, out_v, sg0, sg1, so0, so1):
    w = lax.axis_index("s") * NC + lax.axis_index("c")
    sg = (sg0, sg1)
    so = (so0, so1)

    f = lax.iota(jnp.int32, 16)
    pos_a = ((f >> 3) << 10) + ((f & 7) << 7)

    def idx_load(g, s):
        l0 = g * CL
        pltpu.sync_copy(idx_hbm.at[pl.ds(l0, CL), w, 0], idx0_v.at[s])
        pltpu.sync_copy(idx_hbm.at[pl.ds(l0, CL), w, 1], idx1_v.at[s])

    def gather_start(s):
        for li in range(CL):
            pltpu.async_copy(w0_hbm.at[idx0_v.at[s, li]], rows0_v.at[s, li], sg[s])
            pltpu.async_copy(w1_hbm.at[idx1_v.at[s, li]], rows1_v.at[s, li], sg[s])

    def gather_wait(s):
        for li in range(CL):
            pltpu.make_async_copy(w0_hbm.at[idx0_v.at[s, li]], rows0_v.at[s, li], sg[s]).wait()
            pltpu.make_async_copy(w1_hbm.at[idx1_v.at[s, li]], rows1_v.at[s, li], sg[s]).wait()

    def out_start(l, osl):
        for fh in range(8):
            pltpu.async_copy(out_v.at[osl, pl.ds(fh * 1024, 1024)],
                             out_hbm.at[l, fh, w], so[osl])

    def out_wait(l, osl):
        for fh in range(8):
            pltpu.make_async_copy(out_v.at[osl, pl.ds(fh * 1024, 1024)],
                                  out_hbm.at[l, fh, w], so[osl]).wait()

    def transpose_l(s, li, osl):
        def body(b, carry):
            pos = pos_a + b
            v0 = rows0_v[s, li, b, pl.ds(0, 16)]
            v1 = rows0_v[s, li, b, pl.ds(16, 16)]
            v2 = rows1_v[s, li, b, pl.ds(0, 16)]
            v3 = rows1_v[s, li, b, pl.ds(16, 16)]
            plsc.store_scatter(out_v.at[osl], [pos], v0)
            plsc.store_scatter(out_v.at[osl], [pos + 2048], v1)
            plsc.store_scatter(out_v.at[osl], [pos + 4096], v2)
            plsc.store_scatter(out_v.at[osl], [pos + 6144], v3)
            return carry
        lax.fori_loop(0, 128, body, 0)


    idx_load(0, 0)
    gather_start(0)

    def stage(g, slot):
        nxt = 1 - slot

        @pl.when(g + 1 < GB)
        def _():
            idx_load(g + 1, nxt)
            gather_start(nxt)

        gather_wait(slot)
        for li in range(CL):
            l = g * CL + li
            osl = li % 2

            @pl.when(l >= 2)
            def _():
                out_wait(l - 2, osl)

            transpose_l(slot, li, osl)
            out_start(l, osl)

    def outer(h, carry):
        stage(2 * h, 0)
        stage(2 * h + 1, 1)
        return carry

    lax.fori_loop(0, G2, outer, 0)

    out_wait(L - 2, 0)
    out_wait(L - 1, 1)


def kernel(X, W0, W1):
    idx = X.astype(jnp.int32)
    idx = idx.transpose(1, 0, 2).reshape(L, BT, 128, NVARS).transpose(0, 1, 3, 2)
    out = _embed_sc(idx, W0, W1)
    out = out.reshape(L, 8, BT, 8, 128).transpose(2, 4, 0, 1, 3)
    return out.reshape(B, L, 2 * DIM)


# speedup vs baseline: 1.8430x; 1.8430x over previous
import functools

import jax
import jax.numpy as jnp
from jax import lax
from jax.experimental import pallas as pl
from jax.experimental.pallas import tpu as pltpu
from jax.experimental.pallas import tpu_sc as plsc

B, L, NVARS = 4096, 200, 2
VOCAB, DIM = 1000000, 32
N = B * L

_info = plsc.get_sparse_core_info()
NC, NS = _info.num_cores, _info.num_subcores
NW = NC * NS
BT = B // 128
CL = 4
GB = L // CL
G2 = GB // 2

_mesh = plsc.VectorSubcoreMesh(core_axis_name="c", subcore_axis_name="s")


@functools.partial(
    pl.kernel,
    out_type=jax.ShapeDtypeStruct((L, 8, BT, 1024), jnp.float32),
    mesh=_mesh,
    scratch_types=[
        pltpu.VMEM((2, CL, 128), jnp.int32),
        pltpu.VMEM((2, CL, 128), jnp.int32),
        pltpu.VMEM((2, CL, 128, DIM), jnp.float32),
        pltpu.VMEM((2, CL, 128, DIM), jnp.float32),
        pltpu.VMEM((2, 8192), jnp.float32),
        pltpu.SemaphoreType.DMA,
        pltpu.SemaphoreType.DMA,
        pltpu.SemaphoreType.DMA,
        pltpu.SemaphoreType.DMA,
    ],
    compiler_params=pltpu.CompilerParams(use_tc_tiling_on_sc=False,
                                         needs_layout_passes=False),
)
def _embed_sc(idx_hbm, w0_hbm, w1_hbm, out_hbm,
              idx0_v, idx1_v, rows0_v, rows1_v, out_v, sg0, sg1, so0, so1):
    w = lax.axis_index("s") * NC + lax.axis_index("c")
    sg = (sg0, sg1)
    so = (so0, so1)

    f = lax.iota(jnp.int32, 16)
    pos_a = ((f >> 3) << 10) + ((f & 7) << 7)

    def idx_load(g, s):
        l0 = g * CL
        pltpu.sync_copy(idx_hbm.at[pl.ds(l0, CL), w, 0], idx0_v.at[s])
        pltpu.sync_copy(idx_hbm.at[pl.ds(l0, CL), w, 1], idx1_v.at[s])

    def gather_start(s):
        for li in range(CL):
            pltpu.async_copy(w0_hbm.at[idx0_v.at[s, li]], rows0_v.at[s, li], sg[s])
            pltpu.async_copy(w1_hbm.at[idx1_v.at[s, li]], rows1_v.at[s, li], sg[s])

    def gather_wait(s):
        for li in range(CL):
            pltpu.make_async_copy(w0_hbm.at[idx0_v.at[s, li]], rows0_v.at[s, li], sg[s]).wait()
            pltpu.make_async_copy(w1_hbm.at[idx1_v.at[s, li]], rows1_v.at[s, li], sg[s]).wait()

    def out_start(l, osl):
        for fh in range(8):
            pltpu.async_copy(out_v.at[osl, pl.ds(fh * 1024, 1024)],
                             out_hbm.at[l, fh, w], so[osl])

    def out_wait(l, osl):
        for fh in range(8):
            pltpu.make_async_copy(out_v.at[osl, pl.ds(fh * 1024, 1024)],
                                  out_hbm.at[l, fh, w], so[osl]).wait()

    pos_b = pos_a + 2048
    pos_c = pos_a + 4096
    pos_d = pos_a + 6144

    def transpose_l(s, li, osl):
        @functools.partial(plsc.parallel_loop, 0, 128, unroll=8)
        def _(b):
            v0 = rows0_v[s, li, b, pl.ds(0, 16)]
            v1 = rows0_v[s, li, b, pl.ds(16, 16)]
            v2 = rows1_v[s, li, b, pl.ds(0, 16)]
            v3 = rows1_v[s, li, b, pl.ds(16, 16)]
            plsc.store_scatter(out_v.at[osl], [pos_a + b], v0)
            plsc.store_scatter(out_v.at[osl], [pos_b + b], v1)
            plsc.store_scatter(out_v.at[osl], [pos_c + b], v2)
            plsc.store_scatter(out_v.at[osl], [pos_d + b], v3)


    idx_load(0, 0)
    gather_start(0)

    def stage(g, slot):
        nxt = 1 - slot

        @pl.when(g + 1 < GB)
        def _():
            idx_load(g + 1, nxt)
            gather_start(nxt)

        gather_wait(slot)
        for li in range(CL):
            l = g * CL + li
            osl = li % 2

            @pl.when(l >= 2)
            def _():
                out_wait(l - 2, osl)

            transpose_l(slot, li, osl)
            out_start(l, osl)

    def outer(h, carry):
        stage(2 * h, 0)
        stage(2 * h + 1, 1)
        return carry

    lax.fori_loop(0, G2, outer, 0)

    out_wait(L - 2, 0)
    out_wait(L - 1, 1)


def kernel(X, W0, W1):
    idx = X.astype(jnp.int32)
    idx = idx.transpose(1, 0, 2).reshape(L, BT, 128, NVARS).transpose(0, 1, 3, 2)
    out = _embed_sc(idx, W0, W1)
    out = out.reshape(L, 8, BT, 8, 128).transpose(2, 4, 0, 1, 3)
    return out.reshape(B, L, 2 * DIM)
